# baseline (speedup 1.0000x reference)
import functools

import jax
import jax.numpy as jnp
from jax import lax
from jax.experimental import pallas as pl
from jax.experimental.pallas import tpu as pltpu
from jax.experimental.pallas import tpu_sc as plsc

B, C, H, W = 8, 96, 224, 224
HW = H * W
CHW = C * HW
NPIX = B * HW
NZ = B * CHW

NC, NS, L = 2, 16, 16
NW = NC * NS
P = NPIX // NW
CHUNK = 128
NCHUNK = (2 * P) // CHUNK
NVEC = P // L


def _body(z_hbm, l_hbm, lp_hbm, cond_hbm, out_hbm,
          l_v, lp_v, cond_v, idx_v, val_v, acc_v, sem):
    wid = lax.axis_index("s") * NC + lax.axis_index("c")
    base = wid * P
    b = wid // (NW // B)
    zbase = b * (C - 1) * HW + base

    pltpu.sync_copy(l_hbm.at[pl.ds(base, P)], l_v)
    pltpu.sync_copy(lp_hbm.at[pl.ds(base, P)], lp_v)
    pltpu.sync_copy(cond_hbm.at[pl.ds(base, P)], cond_v)

    iota = lax.iota(jnp.int32, L)

    def compute_idx(i, carry):
        off = zbase + i * L + iota
        lv = l_v[pl.ds(i * L, L)]
        idx_v[pl.ds(i * L, L)] = lv * HW + off
        lpv = lp_v[pl.ds(i * L, L)]
        idx_v[pl.ds(P + i * L, L)] = lpv * HW + off
        return carry

    lax.fori_loop(0, NVEC, compute_idx, 0)

    def fire(j, carry):
        pltpu.async_copy(
            z_hbm.at[idx_v.at[pl.ds(j * CHUNK, CHUNK)]],
            val_v.at[pl.ds(j * CHUNK, CHUNK)],
            sem,
        )
        return carry

    lax.fori_loop(0, NCHUNK, fire, 0)

    pltpu.make_async_copy(z_hbm.at[pl.ds(0, 2 * P)], val_v, sem).wait()

    def accum(i, acc):
        g = val_v[pl.ds(i * L, L)]
        bad = val_v[pl.ds(P + i * L, L)]
        cnd = cond_v[pl.ds(i * L, L)]
        return acc + (g - bad) * cnd

    acc = lax.fori_loop(0, NVEC, accum, jnp.zeros((L,), jnp.float32))
    acc_v[...] = acc
    pltpu.sync_copy(acc_v, out_hbm.at[wid])


@jax.jit
def _loss(z_flat, l_flat, lp_flat, cond_flat):
    k = pl.kernel(
        _body,
        out_type=jax.ShapeDtypeStruct((NW, L), jnp.float32),
        mesh=plsc.VectorSubcoreMesh(core_axis_name="c", subcore_axis_name="s"),
        scratch_types=[
            pltpu.VMEM((P,), jnp.int32),
            pltpu.VMEM((P,), jnp.int32),
            pltpu.VMEM((P,), jnp.float32),
            pltpu.VMEM((2 * P,), jnp.int32),
            pltpu.VMEM((2 * P,), jnp.float32),
            pltpu.VMEM((L,), jnp.float32),
            pltpu.SemaphoreType.DMA,
        ],
    )
    partials = k(z_flat, l_flat, lp_flat, cond_flat)
    return jnp.sum(partials)


def kernel(z, condition, l, l_prime):
    z_flat = z.reshape(-1)
    l_flat = l.reshape(-1).astype(jnp.int32)
    lp_flat = l_prime.reshape(-1).astype(jnp.int32)
    cond_flat = condition.reshape(-1).astype(jnp.float32)
    return _loss(z_flat, l_flat, lp_flat, cond_flat)


# speedup vs baseline: 3.6010x; 3.6010x over previous
import functools

import jax
import jax.numpy as jnp
from jax.experimental import pallas as pl
from jax.experimental.pallas import tpu as pltpu

B, C, H, W = 8, 96, 224, 224
HB = 32
NH = H // HB


def _body(l_ref, lp_ref, cond_ref, z_ref, out_ref):
    lb = l_ref[0]
    lpb = lp_ref[0]
    g = jnp.zeros((HB, W), jnp.float32)
    bad = jnp.zeros((HB, W), jnp.float32)
    for c in range(C):
        zc = z_ref[0, c]
        g = jnp.where(lb == c, zc, g)
        bad = jnp.where(lpb == c, zc, bad)
    out_ref[pl.program_id(0), pl.program_id(1)] = jnp.sum(
        (g - bad) * cond_ref[0]
    )


@jax.jit
def _loss(z, l, lp, cond):
    partials = pl.pallas_call(
        _body,
        grid=(B, NH),
        in_specs=[
            pl.BlockSpec((1, HB, W), lambda b, j: (b, j, 0)),
            pl.BlockSpec((1, HB, W), lambda b, j: (b, j, 0)),
            pl.BlockSpec((1, HB, W), lambda b, j: (b, j, 0)),
            pl.BlockSpec((1, C, HB, W), lambda b, j: (b, 0, j, 0)),
        ],
        out_specs=pl.BlockSpec(
            (B, NH), lambda b, j: (0, 0), memory_space=pltpu.SMEM
        ),
        out_shape=jax.ShapeDtypeStruct((B, NH), jnp.float32),
        compiler_params=pltpu.CompilerParams(
            dimension_semantics=("arbitrary", "arbitrary"),
        ),
    )(l, lp, cond, z)
    return jnp.sum(partials)


def kernel(z, condition, l, l_prime):
    return _loss(
        z,
        l.astype(jnp.int32),
        l_prime.astype(jnp.int32),
        condition.astype(jnp.float32),
    )
